# baseline (speedup 1.0000x reference)
import functools

import jax
import jax.numpy as jnp
from jax import lax
from jax.experimental import pallas as pl
from jax.experimental.pallas import tpu as pltpu
from jax.experimental.pallas import tpu_sc as plsc

V = 1000000
D = 16
B = 16384
L = 50
N = B * L

NC = 2
NS = 16
NW = NC * NS
CB = 1024
NBC = B // CB
ITEMS = L * NBC
PER_W = ITEMS // NW


def _prep(idx_hbm, scale_hbm, idxq_hbm, sub_hbm, scale_c_hbm,
          idx_v, idxs_v, idxq_v, sub_v, scale_v, scale_c_v, sem):
    wid = lax.axis_index("s") * NC + lax.axis_index("c")
    first = wid * PER_W
    iota = lax.iota(jnp.int32, 16)

    def recidx(m, carry):
        val = idx_v[pl.ds(m * 16, 16)]
        idxq_v[pl.ds(m * 16, 16)] = val >> 2
        idxs_v[pl.ds(m * 16, 16)] = val >> 4
        sub_v[pl.ds(m * 16, 16)] = (val & 3) << 2
        return carry

    def scalesel(m, carry):
        val = idx_v[pl.ds(m * 16, 16)]
        scale_c_v[pl.ds(m * 16, 16)] = plsc.load_gather(
            scale_v, [iota + m * 16, val & 15])
        return carry

    def item_loop(t, carry):
        item = first + t
        base = item * CB
        pltpu.sync_copy(idx_hbm.at[pl.ds(base, CB)], idx_v)
        lax.fori_loop(0, CB // 16, recidx, 0, unroll=2)
        pltpu.async_copy(scale_hbm.at[idxs_v], scale_v, sem).wait()
        lax.fori_loop(0, CB // 16, scalesel, 0, unroll=2)
        pltpu.sync_copy(idxq_v, idxq_hbm.at[pl.ds(base, CB)])
        pltpu.sync_copy(sub_v, sub_hbm.at[pl.ds(base, CB)])
        pltpu.sync_copy(scale_c_v, scale_c_hbm.at[pl.ds(base, CB)])
        return carry

    lax.fori_loop(0, PER_W, item_loop, 0)


def _lookup(idxq_hbm, sub_hbm, scale_c_hbm, w_hbm, out_hbm,
            iqa_v, suba_v, sca_v, ra_v,
            iqb_v, subb_v, scb_v, rb_v,
            out_v, gsa, gsb, osem):
    wid = lax.axis_index("s") * NC + lax.axis_index("c")
    first = wid * PER_W

    iota = lax.iota(jnp.int32, 16)
    qiota = iota >> 2
    riota = iota & 3
    lsh = 24 - riota * 8

    gdims = lax.GatherDimensionNumbers(
        offset_dims=(), collapsed_slice_dims=(0,), start_index_map=(0,))

    def stage(item, idxq_v, sub_v, scale_c_v, rows_v, gs):
        base = item * CB
        pltpu.sync_copy(idxq_hbm.at[pl.ds(base, CB)], idxq_v)
        pltpu.sync_copy(sub_hbm.at[pl.ds(base, CB)], sub_v)
        pltpu.sync_copy(scale_c_hbm.at[pl.ds(base, CB)], scale_c_v)
        pltpu.async_copy(w_hbm.at[idxq_v], rows_v, gs)

    def consume(item, idxq_v, sub_v, scale_c_v, rows_v, gs):
        lrow = item >> 4
        bc = item & 15
        pltpu.make_async_copy(w_hbm.at[idxq_v], rows_v, gs).wait()

        @pl.when(item > first)
        def _drain():
            pltpu.make_async_copy(
                out_hbm.at[pl.ds(0, D), pl.ds(0, CB)], out_v, osem).wait()

        def one(k):
            rec = rows_v[k]
            rec32 = plsc.bitcast(rec, jnp.int32)
            sel = plsc.load_gather(sub_v, [iota * 0 + k]) + qiota
            w = lax.gather(rec32, sel[:, None], gdims, (1,),
                           mode=lax.GatherScatterMode.PROMISE_IN_BOUNDS)
            s = plsc.load_gather(scale_c_v, [iota * 0 + k])
            val = ((w << lsh) >> 24).astype(jnp.float32) * s
            plsc.store_scatter(out_v, [iota, iota * 0 + k], val)

        def body(k2, carry):
            one(k2 * 2)
            one(k2 * 2 + 1)
            return carry

        lax.fori_loop(0, CB // 2, body, 0, unroll=2)
        pltpu.async_copy(
            out_v, out_hbm.at[pl.ds(lrow * D, D), pl.ds(bc * CB, CB)], osem)

    bufs_a = (iqa_v, suba_v, sca_v, ra_v, gsa)
    bufs_b = (iqb_v, subb_v, scb_v, rb_v, gsb)

    stage(first, *bufs_a)

    def pair(u, carry):
        base = first + 2 * u
        stage(base + 1, *bufs_b)
        consume(base, *bufs_a)
        stage(base + 2, *bufs_a)
        consume(base + 1, *bufs_b)
        return carry

    lax.fori_loop(0, (PER_W - 1) // 2, pair, 0)
    consume(first + PER_W - 1, *bufs_a)
    pltpu.make_async_copy(
        out_hbm.at[pl.ds(0, D), pl.ds(0, CB)], out_v, osem).wait()


@jax.jit
def _run(idxt, weight, scaleq):
    mesh = plsc.VectorSubcoreMesh(core_axis_name="c", subcore_axis_name="s")
    prep = functools.partial(
        pl.kernel,
        mesh=mesh,
        out_type=(
            jax.ShapeDtypeStruct((N,), jnp.int32),
            jax.ShapeDtypeStruct((N,), jnp.int32),
            jax.ShapeDtypeStruct((N,), jnp.float32),
        ),
        scratch_types=[
            pltpu.VMEM((CB,), jnp.int32),
            pltpu.VMEM((CB,), jnp.int32),
            pltpu.VMEM((CB,), jnp.int32),
            pltpu.VMEM((CB,), jnp.int32),
            pltpu.VMEM((CB, 16), jnp.float32),
            pltpu.VMEM((CB,), jnp.float32),
            pltpu.SemaphoreType.DMA,
        ],
        compiler_params=pltpu.CompilerParams(
            needs_layout_passes=False, use_tc_tiling_on_sc=False),
    )(_prep)
    idxq, sub, scale_c = prep(idxt, scaleq)

    look = functools.partial(
        pl.kernel,
        mesh=mesh,
        out_type=jax.ShapeDtypeStruct((L * D, B), jnp.float32),
        scratch_types=[
            pltpu.VMEM((CB,), jnp.int32),
            pltpu.VMEM((CB,), jnp.int32),
            pltpu.VMEM((CB,), jnp.float32),
            pltpu.VMEM((CB, 64), jnp.int8),
            pltpu.VMEM((CB,), jnp.int32),
            pltpu.VMEM((CB,), jnp.int32),
            pltpu.VMEM((CB,), jnp.float32),
            pltpu.VMEM((CB, 64), jnp.int8),
            pltpu.VMEM((D, CB), jnp.float32),
            pltpu.SemaphoreType.DMA,
            pltpu.SemaphoreType.DMA,
            pltpu.SemaphoreType.DMA,
        ],
        compiler_params=pltpu.CompilerParams(
            needs_layout_passes=False, use_tc_tiling_on_sc=False),
    )(_lookup)
    return look(idxq, sub, scale_c, weight)


def kernel(input, weight, weight_scale):
    idxt = input.T.reshape(-1)
    wrec = weight.reshape(V // 4, 64)
    scaleq = weight_scale.reshape(V // 16, 16)
    out = _run(idxt, wrec, scaleq)
    return out.reshape(L, D, B).transpose(2, 0, 1)


# speedup vs baseline: 1.2280x; 1.2280x over previous
import functools

import jax
import jax.numpy as jnp
from jax import lax
from jax.experimental import pallas as pl
from jax.experimental.pallas import tpu as pltpu
from jax.experimental.pallas import tpu_sc as plsc

V = 1000000
D = 16
B = 16384
L = 50
N = B * L

NC = 2
NS = 16
NW = NC * NS
CB = 1024
NBC = B // CB
ITEMS = L * NBC
PER_W = ITEMS // NW


def _dequant_lookup(idx_hbm, w_hbm, scale_hbm, out_hbm,
                    ia_v, iqa_v, isa_v, ra_v, sca_v, suba_v,
                    ib_v, iqb_v, isb_v, rb_v, scb_v, subb_v,
                    scale_c, out_v, gsa, gsb, osem):
    wid = lax.axis_index("s") * NC + lax.axis_index("c")
    first = wid * PER_W

    iota = lax.iota(jnp.int32, 16)
    qiota = iota >> 2
    riota = iota & 3
    lsh = 24 - riota * 8

    gdims = lax.GatherDimensionNumbers(
        offset_dims=(), collapsed_slice_dims=(0,), start_index_map=(0,))

    def stage(item, idx_v, idxq_v, idxs_v, rows_v, scale_v, sub_v, gs):
        lrow = item >> 4
        bc = item & 15
        base = lrow * B + bc * CB
        pltpu.sync_copy(idx_hbm.at[pl.ds(base, CB)], idx_v)

        def recidx(m, carry):
            val = idx_v[pl.ds(m * 16, 16)]
            idxq_v[pl.ds(m * 16, 16)] = val >> 2
            idxs_v[pl.ds(m * 16, 16)] = val >> 4
            sub_v[pl.ds(m * 16, 16)] = (val & 3) << 2
            return carry

        lax.fori_loop(0, CB // 16, recidx, 0, unroll=2)
        pltpu.async_copy(w_hbm.at[idxq_v], rows_v, gs)
        pltpu.async_copy(scale_hbm.at[idxs_v], scale_v, gs)

    def consume(item, idx_v, idxq_v, idxs_v, rows_v, scale_v, sub_v, gs):
        lrow = item >> 4
        bc = item & 15
        pltpu.make_async_copy(w_hbm.at[idxq_v], rows_v, gs).wait()
        pltpu.make_async_copy(scale_hbm.at[idxs_v], scale_v, gs).wait()

        def scalesel(m, carry):
            val = idx_v[pl.ds(m * 16, 16)]
            scale_c[pl.ds(m * 16, 16)] = plsc.load_gather(
                scale_v, [iota + m * 16, val & 15])
            return carry

        lax.fori_loop(0, CB // 16, scalesel, 0, unroll=2)

        @pl.when(item > first)
        def _drain():
            pltpu.make_async_copy(
                out_hbm.at[pl.ds(0, D), pl.ds(0, CB)], out_v, osem).wait()

        def one(k):
            rec = rows_v[k]
            rec32 = plsc.bitcast(rec, jnp.int32)
            sel = plsc.load_gather(sub_v, [iota * 0 + k]) + qiota
            w = lax.gather(rec32, sel[:, None], gdims, (1,),
                           mode=lax.GatherScatterMode.PROMISE_IN_BOUNDS)
            s = plsc.load_gather(scale_c, [iota * 0 + k])
            val = ((w << lsh) >> 24).astype(jnp.float32) * s
            plsc.store_scatter(out_v, [iota, iota * 0 + k], val)

        @plsc.parallel_loop(0, CB, unroll=4)
        def _body(k):
            one(k)
        pltpu.async_copy(
            out_v, out_hbm.at[pl.ds(lrow * D, D), pl.ds(bc * CB, CB)], osem)

    bufs_a = (ia_v, iqa_v, isa_v, ra_v, sca_v, suba_v, gsa)
    bufs_b = (ib_v, iqb_v, isb_v, rb_v, scb_v, subb_v, gsb)

    stage(first, *bufs_a)

    def pair(u, carry):
        base = first + 2 * u
        stage(base + 1, *bufs_b)
        consume(base, *bufs_a)
        stage(base + 2, *bufs_a)
        consume(base + 1, *bufs_b)
        return carry

    lax.fori_loop(0, (PER_W - 1) // 2, pair, 0)
    consume(first + PER_W - 1, *bufs_a)
    pltpu.make_async_copy(
        out_hbm.at[pl.ds(0, D), pl.ds(0, CB)], out_v, osem).wait()


@jax.jit
def _run(idxt, weight, scaleq):
    mesh = plsc.VectorSubcoreMesh(core_axis_name="c", subcore_axis_name="s")
    f = functools.partial(
        pl.kernel,
        mesh=mesh,
        out_type=jax.ShapeDtypeStruct((L * D, B), jnp.float32),
        scratch_types=[
            pltpu.VMEM((CB,), jnp.int32),
            pltpu.VMEM((CB,), jnp.int32),
            pltpu.VMEM((CB,), jnp.int32),
            pltpu.VMEM((CB, 64), jnp.int8),
            pltpu.VMEM((CB, 16), jnp.float32),
            pltpu.VMEM((CB,), jnp.int32),
            pltpu.VMEM((CB,), jnp.int32),
            pltpu.VMEM((CB,), jnp.int32),
            pltpu.VMEM((CB,), jnp.int32),
            pltpu.VMEM((CB, 64), jnp.int8),
            pltpu.VMEM((CB, 16), jnp.float32),
            pltpu.VMEM((CB,), jnp.int32),
            pltpu.VMEM((CB,), jnp.float32),
            pltpu.VMEM((D, CB), jnp.float32),
            pltpu.SemaphoreType.DMA,
            pltpu.SemaphoreType.DMA,
            pltpu.SemaphoreType.DMA,
        ],
        compiler_params=pltpu.CompilerParams(
            needs_layout_passes=False, use_tc_tiling_on_sc=False),
    )(_dequant_lookup)
    return f(idxt, weight, scaleq)


def kernel(input, weight, weight_scale):
    idxt = input.T.reshape(-1)
    wrec = weight.reshape(V // 4, 64)
    scaleq = weight_scale.reshape(V // 16, 16)
    out = _run(idxt, wrec, scaleq)
    return out.reshape(L, D, B).transpose(2, 0, 1)


# speedup vs baseline: 1.2529x; 1.0203x over previous
import functools

import jax
import jax.numpy as jnp
from jax import lax
from jax.experimental import pallas as pl
from jax.experimental.pallas import tpu as pltpu
from jax.experimental.pallas import tpu_sc as plsc

V = 1000000
D = 16
B = 16384
L = 50
N = B * L

NC = 2
NS = 16
NW = NC * NS
CB = 1024
NBC = B // CB
ITEMS = L * NBC
PER_W = ITEMS // NW


def _dequant_lookup(idx_hbm, w_hbm, scale_hbm, out_hbm,
                    ia_v, iqa_v, isa_v, ra_v, sca_v, suba_v,
                    ib_v, iqb_v, isb_v, rb_v, scb_v, subb_v,
                    scale_c, out_v, gsa, gsb, osem):
    wid = lax.axis_index("s") * NC + lax.axis_index("c")
    first = wid * PER_W

    iota = lax.iota(jnp.int32, 16)
    qiota = iota >> 2
    riota = iota & 3
    lsh = 24 - riota * 8

    gdims = lax.GatherDimensionNumbers(
        offset_dims=(), collapsed_slice_dims=(0,), start_index_map=(0,))

    def stage(item, idx_v, idxq_v, idxs_v, rows_v, scale_v, sub_v, gs):
        lrow = item >> 4
        bc = item & 15
        base = lrow * B + bc * CB
        pltpu.sync_copy(idx_hbm.at[pl.ds(base, CB)], idx_v)

        @plsc.parallel_loop(0, CB // 16, unroll=2)
        def _recidx(m):
            val = idx_v[pl.ds(m * 16, 16)]
            idxq_v[pl.ds(m * 16, 16)] = val >> 2
            idxs_v[pl.ds(m * 16, 16)] = val >> 4
            sub_v[pl.ds(m * 16, 16)] = (val & 3) << 2
        pltpu.async_copy(w_hbm.at[idxq_v], rows_v, gs)
        pltpu.async_copy(scale_hbm.at[idxs_v], scale_v, gs)

    def consume(item, idx_v, idxq_v, idxs_v, rows_v, scale_v, sub_v, gs):
        lrow = item >> 4
        bc = item & 15
        pltpu.make_async_copy(w_hbm.at[idxq_v], rows_v, gs).wait()
        pltpu.make_async_copy(scale_hbm.at[idxs_v], scale_v, gs).wait()

        @plsc.parallel_loop(0, CB // 16, unroll=2)
        def _scalesel(m):
            val = idx_v[pl.ds(m * 16, 16)]
            scale_c[pl.ds(m * 16, 16)] = plsc.load_gather(
                scale_v, [iota + m * 16, val & 15])

        @pl.when(item > first)
        def _drain():
            pltpu.make_async_copy(
                out_hbm.at[pl.ds(0, D), pl.ds(0, CB)], out_v, osem).wait()

        def one(k):
            rec = rows_v[k]
            rec32 = plsc.bitcast(rec, jnp.int32)
            sel = plsc.load_gather(sub_v, [iota * 0 + k]) + qiota
            w = lax.gather(rec32, sel[:, None], gdims, (1,),
                           mode=lax.GatherScatterMode.PROMISE_IN_BOUNDS)
            s = plsc.load_gather(scale_c, [iota * 0 + k])
            val = ((w << lsh) >> 24).astype(jnp.float32) * s
            plsc.store_scatter(out_v, [iota, iota * 0 + k], val)

        @plsc.parallel_loop(0, CB, unroll=4)
        def _body(k):
            one(k)
        pltpu.async_copy(
            out_v, out_hbm.at[pl.ds(lrow * D, D), pl.ds(bc * CB, CB)], osem)

    bufs_a = (ia_v, iqa_v, isa_v, ra_v, sca_v, suba_v, gsa)
    bufs_b = (ib_v, iqb_v, isb_v, rb_v, scb_v, subb_v, gsb)

    stage(first, *bufs_a)

    def pair(u, carry):
        base = first + 2 * u
        stage(base + 1, *bufs_b)
        consume(base, *bufs_a)
        stage(base + 2, *bufs_a)
        consume(base + 1, *bufs_b)
        return carry

    lax.fori_loop(0, (PER_W - 1) // 2, pair, 0)
    consume(first + PER_W - 1, *bufs_a)
    pltpu.make_async_copy(
        out_hbm.at[pl.ds(0, D), pl.ds(0, CB)], out_v, osem).wait()


@jax.jit
def _run(idxt, weight, scaleq):
    mesh = plsc.VectorSubcoreMesh(core_axis_name="c", subcore_axis_name="s")
    f = functools.partial(
        pl.kernel,
        mesh=mesh,
        out_type=jax.ShapeDtypeStruct((L * D, B), jnp.float32),
        scratch_types=[
            pltpu.VMEM((CB,), jnp.int32),
            pltpu.VMEM((CB,), jnp.int32),
            pltpu.VMEM((CB,), jnp.int32),
            pltpu.VMEM((CB, 64), jnp.int8),
            pltpu.VMEM((CB, 16), jnp.float32),
            pltpu.VMEM((CB,), jnp.int32),
            pltpu.VMEM((CB,), jnp.int32),
            pltpu.VMEM((CB,), jnp.int32),
            pltpu.VMEM((CB,), jnp.int32),
            pltpu.VMEM((CB, 64), jnp.int8),
            pltpu.VMEM((CB, 16), jnp.float32),
            pltpu.VMEM((CB,), jnp.int32),
            pltpu.VMEM((CB,), jnp.float32),
            pltpu.VMEM((D, CB), jnp.float32),
            pltpu.SemaphoreType.DMA,
            pltpu.SemaphoreType.DMA,
            pltpu.SemaphoreType.DMA,
        ],
        compiler_params=pltpu.CompilerParams(
            needs_layout_passes=False, use_tc_tiling_on_sc=False),
    )(_dequant_lookup)
    return f(idxt, weight, scaleq)


def kernel(input, weight, weight_scale):
    idxt = input.T.reshape(-1)
    wrec = weight.reshape(V // 4, 64)
    scaleq = weight_scale.reshape(V // 16, 16)
    out = _run(idxt, wrec, scaleq)
    return out.reshape(L, D, B).transpose(2, 0, 1)
